# baseline (speedup 1.0000x reference)
import jax
import jax.numpy as jnp
from jax import lax
from jax.experimental import pallas as pl
from jax.experimental.pallas import tpu as pltpu
from jax.experimental.pallas import tpu_sc as plsc

N_NODES = 1000000
EMBED_DIM = 64
BATCH = 16384

NUM_CORES = 2
NUM_SUBCORES = 16
LANES = 16
NUM_WORKERS = NUM_CORES * NUM_SUBCORES

ROWS_PER_WORKER = BATCH // NUM_WORKERS
BLK = 128
GROUP = 8
NUM_GROUPS = ROWS_PER_WORKER // GROUP


def _rsqrt_nr(s):
    i = plsc.bitcast(s, jnp.int32)
    i = jnp.int32(0x5F3759DF) - lax.shift_right_arithmetic(i, jnp.int32(1))
    y = plsc.bitcast(i, jnp.float32)
    for _ in range(3):
        y = y * (1.5 - 0.5 * s * y * y)
    return y


def _sc_body(nodes_hbm, tablet_hbm, outt_hbm, idx_v, blk_v, buft_v, tmp_v,
             *sems):
    wid = lax.axis_index("s") * NUM_CORES + lax.axis_index("c")
    base = wid * ROWS_PER_WORKER

    pltpu.sync_copy(nodes_hbm.at[pl.ds(base, ROWS_PER_WORKER)],
                    idx_v.at[pl.ds(0, ROWS_PER_WORKER)])
    idx_v[pl.ds(ROWS_PER_WORKER, LANES)] = jnp.zeros((LANES,), jnp.int32)

    def fire_group(g):
        v = idx_v[pl.ds(g * GROUP, LANES)]
        for t in range(GROUP):
            cw = lax.shift_right_logical(v[t], jnp.int32(7)) * jnp.int32(BLK)
            pltpu.async_copy(
                tablet_hbm.at[:, pl.ds(cw, BLK)],
                blk_v.at[t],
                sems[t])

    fire_group(0)

    lanes = lax.iota(jnp.int32, LANES)
    halves = lax.shift_right_logical(lanes, jnp.int32(1))
    parity = lax.bitwise_and(lanes, jnp.int32(1))

    def wave(g, carry):
        for t in range(GROUP):
            pltpu.make_async_copy(
                tablet_hbm.at[:, pl.ds(0, BLK)], blk_v.at[t], sems[t]).wait()

        ids = plsc.load_gather(idx_v, [g * GROUP + halves])
        q = lax.bitwise_and(ids, jnp.int32(BLK - 1))

        acc = jnp.zeros((LANES,), jnp.float32)
        xs = []
        for k in range(EMBED_DIM // 2):
            jv = jnp.full((LANES,), 2 * k, jnp.int32) + parity
            x = plsc.load_gather(blk_v, [halves, jv, q])
            xs.append((jv, x))
            acc = acc + x * x

        @pl.when(g < NUM_GROUPS - 1)
        def _():
            fire_group(g + 1)

        tmp_v[...] = acc
        acc = acc + plsc.load_gather(tmp_v, [lax.bitwise_xor(lanes,
                                                             jnp.int32(1))])
        r = _rsqrt_nr(jnp.maximum(acc, jnp.float32(1e-24)))

        cols = g * GROUP + halves
        for jv, x in xs:
            plsc.store_scatter(buft_v, [jv, cols], x * r)

        return carry

    lax.fori_loop(0, NUM_GROUPS, wave, 0)

    pltpu.sync_copy(buft_v, outt_hbm.at[:, pl.ds(base, ROWS_PER_WORKER)])


@jax.jit
def _pale_embedding_sc(nodes, table):
    mesh = plsc.VectorSubcoreMesh(core_axis_name="c", subcore_axis_name="s")
    outt = pl.kernel(
        _sc_body,
        out_type=jax.ShapeDtypeStruct((EMBED_DIM, BATCH), jnp.float32),
        mesh=mesh,
        scratch_types=[
            pltpu.VMEM((ROWS_PER_WORKER + LANES,), jnp.int32),
            pltpu.VMEM((GROUP, EMBED_DIM, BLK), jnp.float32),
            pltpu.VMEM((EMBED_DIM, ROWS_PER_WORKER), jnp.float32),
            pltpu.VMEM((LANES,), jnp.float32),
        ] + [pltpu.SemaphoreType.DMA] * GROUP,
        compiler_params=pltpu.CompilerParams(needs_layout_passes=False),
    )(nodes, table.T)
    return outt.T


def kernel(nodes, table):
    return _pale_embedding_sc(nodes, table)


# speedup vs baseline: 1.0825x; 1.0825x over previous
import jax
import jax.numpy as jnp
from jax import lax
from jax.experimental import pallas as pl
from jax.experimental.pallas import tpu as pltpu
from jax.experimental.pallas import tpu_sc as plsc

N_NODES = 1000000
EMBED_DIM = 64
BATCH = 16384

NUM_CORES = 2
NUM_SUBCORES = 16
LANES = 16
NUM_WORKERS = NUM_CORES * NUM_SUBCORES

ROWS_PER_WORKER = BATCH // NUM_WORKERS
BLK = 128
GROUP = 8
NUM_GROUPS = ROWS_PER_WORKER // GROUP


def _rsqrt_nr(s):
    i = plsc.bitcast(s, jnp.int32)
    i = jnp.int32(0x5F3759DF) - lax.shift_right_arithmetic(i, jnp.int32(1))
    y = plsc.bitcast(i, jnp.float32)
    for _ in range(3):
        y = y * (1.5 - 0.5 * s * y * y)
    return y


def _sc_body(nodes_hbm, tablet_hbm, outt_hbm, idx_v, blk_v, buft_v, tmp_v,
             *sems):
    wid = lax.axis_index("s") * NUM_CORES + lax.axis_index("c")
    base = wid * ROWS_PER_WORKER

    pltpu.sync_copy(nodes_hbm.at[pl.ds(base, ROWS_PER_WORKER)],
                    idx_v.at[pl.ds(0, ROWS_PER_WORKER)])
    idx_v[pl.ds(ROWS_PER_WORKER, LANES)] = jnp.zeros((LANES,), jnp.int32)

    def fire_group(g):
        v = idx_v[pl.ds(g * GROUP, LANES)]
        for t in range(GROUP):
            cw = lax.shift_right_logical(v[t], jnp.int32(7)) * jnp.int32(BLK)
            pltpu.async_copy(
                tablet_hbm.at[:, pl.ds(cw, BLK)],
                blk_v.at[t],
                sems[0])

    fire_group(0)

    lanes = lax.iota(jnp.int32, LANES)
    halves = lax.shift_right_logical(lanes, jnp.int32(1))
    parity = lax.bitwise_and(lanes, jnp.int32(1))

    def wave(g, carry):
        for _ in range(2):
            pltpu.make_async_copy(
                tablet_hbm.at[:, pl.ds(0, ROWS_PER_WORKER)], buft_v,
                sems[0]).wait()

        ids = plsc.load_gather(idx_v, [g * GROUP + halves])
        q = lax.bitwise_and(ids, jnp.int32(BLK - 1))

        acc = jnp.zeros((LANES,), jnp.float32)
        xs = []
        for k in range(EMBED_DIM // 2):
            jv = jnp.full((LANES,), 2 * k, jnp.int32) + parity
            x = plsc.load_gather(blk_v, [halves, jv, q])
            xs.append((jv, x))
            acc = acc + x * x

        @pl.when(g < NUM_GROUPS - 1)
        def _():
            fire_group(g + 1)

        tmp_v[...] = acc
        acc = acc + plsc.load_gather(tmp_v, [lax.bitwise_xor(lanes,
                                                             jnp.int32(1))])
        r = _rsqrt_nr(jnp.maximum(acc, jnp.float32(1e-24)))

        cols = g * GROUP + halves
        for jv, x in xs:
            plsc.store_scatter(buft_v, [jv, cols], x * r)

        return carry

    lax.fori_loop(0, NUM_GROUPS, wave, 0)

    pltpu.sync_copy(buft_v, outt_hbm.at[:, pl.ds(base, ROWS_PER_WORKER)])


@jax.jit
def _pale_embedding_sc(nodes, table):
    mesh = plsc.VectorSubcoreMesh(core_axis_name="c", subcore_axis_name="s")
    outt = pl.kernel(
        _sc_body,
        out_type=jax.ShapeDtypeStruct((EMBED_DIM, BATCH), jnp.float32),
        mesh=mesh,
        scratch_types=[
            pltpu.VMEM((ROWS_PER_WORKER + LANES,), jnp.int32),
            pltpu.VMEM((GROUP, EMBED_DIM, BLK), jnp.float32),
            pltpu.VMEM((EMBED_DIM, ROWS_PER_WORKER), jnp.float32),
            pltpu.VMEM((LANES,), jnp.float32),
        ] + [pltpu.SemaphoreType.DMA],
        compiler_params=pltpu.CompilerParams(needs_layout_passes=False),
    )(nodes, table.T)
    return outt.T


def kernel(nodes, table):
    return _pale_embedding_sc(nodes, table)
